# baseline (speedup 1.0000x reference)
import functools

import jax
import jax.numpy as jnp
from jax.experimental import pallas as pl
from jax.experimental.pallas import tpu as pltpu

N = 32768
DIM = 2048
HID = 128
K = 64
TILE = 2048
GRID = N // TILE
SROWS = N // 128
VREGS = SROWS // 8
INT_MIN = -(1 << 31)


def _vreg_tree(arr, op):
    parts = [arr[8 * g:8 * g + 8, :] for g in range(VREGS)]
    while len(parts) > 1:
        parts = [op(parts[2 * i], parts[2 * i + 1]) for i in range(len(parts) // 2)]
    return parts[0]


def _sublane_sum(v):
    for sh in (1, 2, 4):
        v = v + pltpu.roll(v, sh, 0)
    return v


def _lane_splat_sum(v):
    v = _sublane_sum(v)
    for sh in (1, 2, 4, 8, 16, 32, 64):
        v = v + pltpu.roll(v, sh, 1)
    return v


def _fused_kernel(x0_ref, x1_ref, x2_ref, x3_ref, w1_ref, b1_ref, w2_ref, b2_ref, x_hbm,
                  out_ref, scores_ref, keys_ref, ones_ref, lt_ref,
                  idxv_ref, idxs_ref, rows_ref, sem, sem2):
    i = pl.program_id(0)

    Q = DIM // 4
    h = jnp.dot(x0_ref[...], w1_ref[0:Q, :], preferred_element_type=jnp.float32)
    h = h + jnp.dot(x1_ref[...], w1_ref[Q:2 * Q, :], preferred_element_type=jnp.float32)
    h = h + jnp.dot(x2_ref[...], w1_ref[2 * Q:3 * Q, :], preferred_element_type=jnp.float32)
    h = h + jnp.dot(x3_ref[...], w1_ref[3 * Q:4 * Q, :], preferred_element_type=jnp.float32)
    h = jnp.maximum(h + b1_ref[...], 0.0)
    w = jnp.dot(h, w2_ref[...], preferred_element_type=jnp.float32)
    w = w + b2_ref[0, 0]
    tile_rows = TILE // 128
    scores_ref[pl.ds(i * tile_rows, tile_rows), :] = w.reshape(tile_rows, 128)

    @pl.when(i == GRID - 1)
    def _():
        lane_i = jax.lax.broadcasted_iota(jnp.int32, (128, 128), 0)
        lane_j = jax.lax.broadcasted_iota(jnp.int32, (128, 128), 1)
        ones_ref[...] = jnp.ones((128, 128), jnp.float32)
        lt_ref[...] = (lane_i < lane_j).astype(jnp.float32)

        u = jax.lax.bitcast_convert_type(scores_ref[...], jnp.int32)
        keys_ref[...] = jnp.where(u >= 0, u, u ^ jnp.int32(0x7FFFFFFF))

        def total_splat(b):
            t = _vreg_tree(b, jnp.add)
            t = jnp.dot(t, ones_ref[...], preferred_element_type=jnp.float32)
            return _sublane_sum(t)

        def count_ge(cand):
            b = jnp.where(keys_ref[...] >= cand, 1.0, 0.0)
            return total_splat(b)[0:1, 0:1]

        zero = jnp.zeros((1, 1), jnp.int32)
        p = jnp.where(count_ge(zero) >= K, zero,
                      jnp.full((1, 1), INT_MIN, jnp.int32))
        for t in range(31):
            cand = p | (1 << (30 - t))
            p = jnp.where(count_ge(cand) >= K, cand, p)
        tau = p

        keys = keys_ref[...]
        strict = keys > tau
        tie = keys == tau
        g_cnt = count_ge(tau + 1)
        t_need = K - g_cnt

        def excl_prefix(mask_f32):
            lanep = jnp.dot(mask_f32, lt_ref[...],
                            preferred_element_type=jnp.float32)
            rowtot = jnp.dot(mask_f32, ones_ref[...],
                             preferred_element_type=jnp.float32)
            sub = jax.lax.broadcasted_iota(jnp.int32, (8, 128), 0)
            outs = []
            carry = jnp.zeros((8, 128), jnp.float32)
            for g in range(VREGS):
                v = rowtot[8 * g:8 * g + 8, :]
                incl = v
                for sh in (1, 2, 4):
                    r = pltpu.roll(incl, sh, 0)
                    incl = incl + jnp.where(sub >= sh, r, 0.0)
                excl = incl - v
                outs.append(excl + carry)
                tot = pltpu.roll(incl, 1, 0)
                for sh in (1, 2, 4):
                    r = pltpu.roll(tot, sh, 0)
                    tot = jnp.where(sub >= sh, r, tot)
                carry = carry + tot
            rowp = jnp.concatenate(outs, axis=0)
            return lanep + rowp

        tie_f = jnp.where(tie, 1.0, 0.0)
        tie_rank = excl_prefix(tie_f)
        sel = strict | (tie & (tie_rank < t_need))
        sel_f = jnp.where(sel, 1.0, 0.0)
        rank = excl_prefix(sel_f)

        iota = jax.lax.broadcasted_iota(jnp.int32, (SROWS, 128), 0) * 128 + \
            jax.lax.broadcasted_iota(jnp.int32, (SROWS, 128), 1)
        slot = jnp.where(sel, rank.astype(jnp.int32), -1)
        zval = jnp.where(sel, iota.astype(jnp.float32), 0.0)

        pos = jax.lax.broadcasted_iota(jnp.int32, (8, 128), 1) + \
            jax.lax.broadcasted_iota(jnp.int32, (8, 128), 0) * 1024
        idx_acc = jnp.zeros((8, 128), jnp.float32)
        for j in range(K):
            zj = jnp.where(slot == j, zval, 0.0)
            sj = _lane_splat_sum(_vreg_tree(zj, jnp.add))
            idx_acc = jnp.where(pos == j, sj, idx_acc)

        idxv_ref[...] = idx_acc[0:1, :].astype(jnp.int32)
        pltpu.sync_copy(idxv_ref, idxs_ref)

        def dma_body(j, carry):
            row = idxs_ref[0, j]
            pltpu.make_async_copy(
                x_hbm.at[pl.ds(row, 1)], rows_ref.at[pl.ds(j, 1)], sem
            ).start()
            return carry

        jax.lax.fori_loop(0, K, dma_body, 0)
        pltpu.make_async_copy(x_hbm.at[pl.ds(0, K)], rows_ref, sem).wait()
        out_ref[...] = jnp.sum(rows_ref[...], axis=0, keepdims=True) * (1.0 / K)


@jax.jit
def kernel(x, W1, b1, W2, b2):
    out = pl.pallas_call(
        _fused_kernel,
        grid=(GRID,),
        in_specs=[
            pl.BlockSpec((TILE, DIM // 4), lambda i: (i, 0)),
            pl.BlockSpec((TILE, DIM // 4), lambda i: (i, 1)),
            pl.BlockSpec((TILE, DIM // 4), lambda i: (i, 2)),
            pl.BlockSpec((TILE, DIM // 4), lambda i: (i, 3)),
            pl.BlockSpec((DIM, HID), lambda i: (0, 0)),
            pl.BlockSpec((1, HID), lambda i: (0, 0)),
            pl.BlockSpec((HID, 1), lambda i: (0, 0)),
            pl.BlockSpec((1, 1), lambda i: (0, 0)),
            pl.BlockSpec(memory_space=pltpu.MemorySpace.HBM),
        ],
        out_specs=pl.BlockSpec((1, DIM), lambda i: (0, 0)),
        out_shape=jax.ShapeDtypeStruct((1, DIM), jnp.float32),
        scratch_shapes=[
            pltpu.VMEM((SROWS, 128), jnp.float32),
            pltpu.VMEM((SROWS, 128), jnp.int32),
            pltpu.VMEM((128, 128), jnp.float32),
            pltpu.VMEM((128, 128), jnp.float32),
            pltpu.VMEM((1, 128), jnp.int32),
            pltpu.SMEM((1, 128), jnp.int32),
            pltpu.VMEM((K, DIM), jnp.float32),
            pltpu.SemaphoreType.DMA,
            pltpu.SemaphoreType.DMA,
        ],
    )(x, x, x, x, W1, b1.reshape(1, HID), W2, b2.reshape(1, 1), x)
    return out.reshape(DIM)


# speedup vs baseline: 1.0063x; 1.0063x over previous
import functools

import jax
import jax.numpy as jnp
from jax.experimental import pallas as pl
from jax.experimental.pallas import tpu as pltpu

N = 32768
DIM = 2048
HID = 128
K = 64
TILE = 2048
GRID = N // TILE
SROWS = N // 128
VREGS = SROWS // 8
INT_MIN = -(1 << 31)


def _vreg_tree(arr, op):
    parts = [arr[8 * g:8 * g + 8, :] for g in range(VREGS)]
    while len(parts) > 1:
        parts = [op(parts[2 * i], parts[2 * i + 1]) for i in range(len(parts) // 2)]
    return parts[0]


def _sublane_sum(v):
    for sh in (1, 2, 4):
        v = v + pltpu.roll(v, sh, 0)
    return v


def _lane_splat_sum(v):
    v = _sublane_sum(v)
    for sh in (1, 2, 4, 8, 16, 32, 64):
        v = v + pltpu.roll(v, sh, 1)
    return v


def _fused_kernel(xl_ref, xr_ref, w1_ref, b1_ref, w2_ref, b2_ref, x_hbm,
                  out_ref, scores_ref, keys_ref, ones_ref, lt_ref,
                  idxv_ref, idxs_ref, rows_ref, sem, sem2):
    i = pl.program_id(0)

    HALF = DIM // 2
    h = jnp.dot(xl_ref[...], w1_ref[0:HALF, :], preferred_element_type=jnp.float32)
    h = h + jnp.dot(xr_ref[...], w1_ref[HALF:DIM, :], preferred_element_type=jnp.float32)
    h = jnp.maximum(h + b1_ref[...], 0.0)
    w = jnp.dot(h, w2_ref[...], preferred_element_type=jnp.float32)
    w = w + b2_ref[0, 0]
    tile_rows = TILE // 128
    scores_ref[pl.ds(i * tile_rows, tile_rows), :] = w.reshape(tile_rows, 128)

    @pl.when(i == GRID - 1)
    def _():
        lane_i = jax.lax.broadcasted_iota(jnp.int32, (128, 128), 0)
        lane_j = jax.lax.broadcasted_iota(jnp.int32, (128, 128), 1)
        ones_ref[...] = jnp.ones((128, 128), jnp.float32)
        lt_ref[...] = (lane_i < lane_j).astype(jnp.float32)

        u = jax.lax.bitcast_convert_type(scores_ref[...], jnp.int32)
        keys_ref[...] = jnp.where(u >= 0, u, u ^ jnp.int32(0x7FFFFFFF))

        def total_splat(b):
            t = _vreg_tree(b, jnp.add)
            t = jnp.dot(t, ones_ref[...], preferred_element_type=jnp.float32)
            return _sublane_sum(t)

        def count_ge(cand):
            b = jnp.where(keys_ref[...] >= cand, 1.0, 0.0)
            return total_splat(b)[0:1, 0:1]

        zero = jnp.zeros((1, 1), jnp.int32)
        p = jnp.where(count_ge(zero) >= K, zero,
                      jnp.full((1, 1), INT_MIN, jnp.int32))
        for t in range(31):
            cand = p | (1 << (30 - t))
            p = jnp.where(count_ge(cand) >= K, cand, p)
        tau = p

        keys = keys_ref[...]
        strict = keys > tau
        tie = keys == tau
        g_cnt = count_ge(tau + 1)
        t_need = K - g_cnt

        def excl_prefix(mask_f32):
            lanep = jnp.dot(mask_f32, lt_ref[...],
                            preferred_element_type=jnp.float32)
            rowtot = jnp.dot(mask_f32, ones_ref[...],
                             preferred_element_type=jnp.float32)
            sub = jax.lax.broadcasted_iota(jnp.int32, (8, 128), 0)
            outs = []
            carry = jnp.zeros((8, 128), jnp.float32)
            for g in range(VREGS):
                v = rowtot[8 * g:8 * g + 8, :]
                incl = v
                for sh in (1, 2, 4):
                    r = pltpu.roll(incl, sh, 0)
                    incl = incl + jnp.where(sub >= sh, r, 0.0)
                excl = incl - v
                outs.append(excl + carry)
                tot = pltpu.roll(incl, 1, 0)
                for sh in (1, 2, 4):
                    r = pltpu.roll(tot, sh, 0)
                    tot = jnp.where(sub >= sh, r, tot)
                carry = carry + tot
            rowp = jnp.concatenate(outs, axis=0)
            return lanep + rowp

        tie_f = jnp.where(tie, 1.0, 0.0)
        tie_rank = excl_prefix(tie_f)
        sel = strict | (tie & (tie_rank < t_need))
        sel_f = jnp.where(sel, 1.0, 0.0)
        rank = excl_prefix(sel_f)

        iota = jax.lax.broadcasted_iota(jnp.int32, (SROWS, 128), 0) * 128 + \
            jax.lax.broadcasted_iota(jnp.int32, (SROWS, 128), 1)
        slot = jnp.where(sel, rank.astype(jnp.int32), -1)
        zval = jnp.where(sel, iota.astype(jnp.float32), 0.0)

        pos = jax.lax.broadcasted_iota(jnp.int32, (8, 128), 1) + \
            jax.lax.broadcasted_iota(jnp.int32, (8, 128), 0) * 1024
        idx_acc = jnp.zeros((8, 128), jnp.float32)
        for j in range(K):
            zj = jnp.where(slot == j, zval, 0.0)
            sj = _lane_splat_sum(_vreg_tree(zj, jnp.add))
            idx_acc = jnp.where(pos == j, sj, idx_acc)

        idxv_ref[...] = idx_acc[0:1, :].astype(jnp.int32)
        pltpu.sync_copy(idxv_ref, idxs_ref)

        def dma_body(j, carry):
            row = idxs_ref[0, j]
            pltpu.make_async_copy(
                x_hbm.at[pl.ds(row, 1)], rows_ref.at[pl.ds(j, 1)], sem
            ).start()
            return carry

        jax.lax.fori_loop(0, K, dma_body, 0)
        pltpu.make_async_copy(x_hbm.at[pl.ds(0, K)], rows_ref, sem).wait()
        out_ref[...] = jnp.sum(rows_ref[...], axis=0, keepdims=True) * (1.0 / K)


@jax.jit
def kernel(x, W1, b1, W2, b2):
    out = pl.pallas_call(
        _fused_kernel,
        grid=(GRID,),
        in_specs=[
            pl.BlockSpec((TILE, DIM // 2), lambda i: (i, 0)),
            pl.BlockSpec((TILE, DIM // 2), lambda i: (i, 1)),
            pl.BlockSpec((DIM, HID), lambda i: (0, 0)),
            pl.BlockSpec((1, HID), lambda i: (0, 0)),
            pl.BlockSpec((HID, 1), lambda i: (0, 0)),
            pl.BlockSpec((1, 1), lambda i: (0, 0)),
            pl.BlockSpec(memory_space=pltpu.MemorySpace.HBM),
        ],
        out_specs=pl.BlockSpec((1, DIM), lambda i: (0, 0)),
        out_shape=jax.ShapeDtypeStruct((1, DIM), jnp.float32),
        scratch_shapes=[
            pltpu.VMEM((SROWS, 128), jnp.float32),
            pltpu.VMEM((SROWS, 128), jnp.int32),
            pltpu.VMEM((128, 128), jnp.float32),
            pltpu.VMEM((128, 128), jnp.float32),
            pltpu.VMEM((1, 128), jnp.int32),
            pltpu.SMEM((1, 128), jnp.int32),
            pltpu.VMEM((K, DIM), jnp.float32),
            pltpu.SemaphoreType.DMA,
            pltpu.SemaphoreType.DMA,
        ],
    )(x, x, W1, b1.reshape(1, HID), W2, b2.reshape(1, 1), x)
    return out.reshape(DIM)
